# baseline (speedup 1.0000x reference)
import jax
import jax.numpy as jnp
from jax import lax
from jax.experimental import pallas as pl
from jax.experimental.pallas import tpu as pltpu
from jax.experimental.pallas import tpu_sc as plsc

B = 16384
DIM = 16
NC = 2
NS = 16
NW = NC * NS
BPW = B // NW
CHUNK = 128
NCH = BPW // CHUNK
NG = BPW // 16


def _sc_body(uid_h, pid_h, nid_h, uemb_h, iemb_h, ubias_h, ibias_h, gb_h,
             pos_h, neg_h,
             uid_v, pid_v, nid_v, urows, prows, nrows,
             ub_v, pb_v, nb_v, gb_v, pos_v, neg_v, sem):
  wid = lax.axis_index("s") * NC + lax.axis_index("c")
  base = wid * BPW

  pltpu.sync_copy(gb_h, gb_v)

  for j in range(NCH):
    sl_h = pl.ds(base + j * CHUNK, CHUNK)
    pltpu.sync_copy(uid_h.at[sl_h], uid_v.at[j])
    pltpu.sync_copy(pid_h.at[sl_h], pid_v.at[j])
    pltpu.sync_copy(nid_h.at[sl_h], nid_v.at[j])

  copies = []
  for j in range(NCH):
    sl = pl.ds(j * CHUNK, CHUNK)
    copies.append(pltpu.make_async_copy(uemb_h.at[uid_v.at[j]], urows.at[sl], sem))
    copies.append(pltpu.make_async_copy(iemb_h.at[pid_v.at[j]], prows.at[sl], sem))
    copies.append(pltpu.make_async_copy(iemb_h.at[nid_v.at[j]], nrows.at[sl], sem))
    copies.append(pltpu.make_async_copy(ubias_h.at[uid_v.at[j]], ub_v.at[sl], sem))
    copies.append(pltpu.make_async_copy(ibias_h.at[pid_v.at[j]], pb_v.at[sl], sem))
    copies.append(pltpu.make_async_copy(ibias_h.at[nid_v.at[j]], nb_v.at[sl], sem))
  for c in copies:
    c.start()
  for c in copies:
    c.wait()

  gbv = gb_v[...]
  iota16 = lax.iota(jnp.int32, 16)
  lane_masks = [iota16 == i for i in range(16)]

  def group(g, carry):
    rbase = g * 16
    ubv = ub_v[pl.ds(rbase, 16)]
    pbv = pb_v[pl.ds(rbase, 16)]
    nbv = nb_v[pl.ds(rbase, 16)]
    pos = gbv + ubv + pbv
    neg = gbv + ubv + nbv
    for i in range(16):
      r = rbase + i
      u = urows[r, :]
      p = prows[r, :]
      n = nrows[r, :]
      dp = jnp.sum(u * p)
      dn = jnp.sum(u * n)
      pos = pos + jnp.where(lane_masks[i], dp, 0.0)
      neg = neg + jnp.where(lane_masks[i], dn, 0.0)
    pos_v[pl.ds(rbase, 16)] = pos
    neg_v[pl.ds(rbase, 16)] = neg
    return carry

  lax.fori_loop(0, NG, group, 0)

  pltpu.sync_copy(pos_v, pos_h.at[pl.ds(base, BPW)])
  pltpu.sync_copy(neg_v, neg_h.at[pl.ds(base, BPW)])


def kernel(user_ids, pos_item_ids, neg_item_ids, user_emb_w, item_emb_w,
           user_bias_w, item_bias_w, global_bias):
  gb16 = jnp.broadcast_to(global_bias.astype(jnp.float32), (16,))
  ubias_flat = user_bias_w.reshape(-1)
  ibias_flat = item_bias_w.reshape(-1)
  mesh = plsc.VectorSubcoreMesh(core_axis_name="c", subcore_axis_name="s",
                                num_cores=NC, num_subcores=NS)
  f = pl.kernel(
      _sc_body,
      out_type=(jax.ShapeDtypeStruct((B,), jnp.float32),
                jax.ShapeDtypeStruct((B,), jnp.float32)),
      mesh=mesh,
      compiler_params=pltpu.CompilerParams(needs_layout_passes=False,
                                           use_tc_tiling_on_sc=False),
      scratch_types=[
          pltpu.VMEM((NCH, CHUNK), jnp.int32),
          pltpu.VMEM((NCH, CHUNK), jnp.int32),
          pltpu.VMEM((NCH, CHUNK), jnp.int32),
          pltpu.VMEM((BPW, DIM), jnp.float32),
          pltpu.VMEM((BPW, DIM), jnp.float32),
          pltpu.VMEM((BPW, DIM), jnp.float32),
          pltpu.VMEM((BPW,), jnp.float32),
          pltpu.VMEM((BPW,), jnp.float32),
          pltpu.VMEM((BPW,), jnp.float32),
          pltpu.VMEM((16,), jnp.float32),
          pltpu.VMEM((BPW,), jnp.float32),
          pltpu.VMEM((BPW,), jnp.float32),
          pltpu.SemaphoreType.DMA,
      ],
  )
  return f(user_ids, pos_item_ids, neg_item_ids, user_emb_w, item_emb_w,
           ubias_flat, ibias_flat, gb16)
